# baseline (speedup 1.0000x reference)
import functools

import jax
import jax.numpy as jnp
from jax import lax
from jax.experimental import pallas as pl
from jax.experimental.pallas import tpu as pltpu
from jax.experimental.pallas import tpu_sc as plsc

_INFO = plsc.get_sparse_core_info()
_NC = _INFO.num_cores
_NS = _INFO.num_subcores
_NW = _NC * _NS
_CHUNK = 128


def _make_lookup(n_rows, dim, batch):
    assert batch % (_NW * _CHUNK) == 0
    b_per_w = batch // _NW
    n_chunks = b_per_w // _CHUNK
    mesh = plsc.VectorSubcoreMesh(core_axis_name="c", subcore_axis_name="s")

    @functools.partial(
        pl.kernel,
        mesh=mesh,
        out_type=jax.ShapeDtypeStruct((batch, dim), jnp.float32),
        scratch_types=[
            pltpu.VMEM((n_chunks, _CHUNK), jnp.int32),
            pltpu.VMEM((b_per_w, dim), jnp.float32),
            pltpu.SemaphoreType.DMA,
        ],
        compiler_params=pltpu.CompilerParams(use_tc_tiling_on_sc=False),
    )
    def lookup(idx_hbm, table_hbm, out_hbm, idx_v, rows_v, sem):
        wid = lax.axis_index("s") * _NC + lax.axis_index("c")
        base = wid * b_per_w
        pltpu.sync_copy(idx_hbm.at[pl.ds(wid * n_chunks, n_chunks)], idx_v)
        copies = []
        for j in range(n_chunks):
            copies.append(
                pltpu.async_copy(
                    table_hbm.at[idx_v.at[j]],
                    rows_v.at[pl.ds(j * _CHUNK, _CHUNK)],
                    sem,
                )
            )
        for c in copies:
            c.wait()
        pltpu.sync_copy(rows_v, out_hbm.at[pl.ds(base, b_per_w)])

    return lookup


@jax.jit
def kernel(skill_id, emb_weight):
    batch = skill_id.shape[0]
    n_rows, dim = emb_weight.shape
    idx2d = skill_id.astype(jnp.int32).reshape(batch // _CHUNK, _CHUNK)
    return _make_lookup(n_rows, dim, batch)(idx2d, emb_weight)


# speedup vs baseline: 3.0066x; 3.0066x over previous
import functools

import jax
import jax.numpy as jnp
from jax import lax
from jax.experimental import pallas as pl
from jax.experimental.pallas import tpu as pltpu
from jax.experimental.pallas import tpu_sc as plsc

_INFO = plsc.get_sparse_core_info()
_NC = _INFO.num_cores
_NS = _INFO.num_subcores
_NW = _NC * _NS
_B = 16
_RING = 8


def _make_lookup(dim, batch):
    assert batch % (_NW * _B) == 0 and batch % (_NW * _RING) == 0
    b_per_w = batch // _NW
    mesh = plsc.VectorSubcoreMesh(core_axis_name="c", subcore_axis_name="s")

    @functools.partial(
        pl.kernel,
        mesh=mesh,
        out_type=jax.ShapeDtypeStruct((dim, batch), jnp.float32),
        scratch_types=[
            pltpu.VMEM((b_per_w + _B,), jnp.int32),
            pltpu.VMEM((_RING, dim, 128), jnp.float32),
            pltpu.VMEM((b_per_w, dim), jnp.float32),
            pltpu.VMEM((dim, b_per_w), jnp.float32),
            pltpu.SemaphoreType.DMA,
        ],
        compiler_params=pltpu.CompilerParams(needs_layout_passes=False),
    )
    def lookup(idx_hbm, tab_hbm, out_hbm, idx_v, ring_v, stage_v, outt_v,
               sem):
        wid = lax.axis_index("s") * _NC + lax.axis_index("c")
        base = wid * b_per_w
        pltpu.sync_copy(
            idx_hbm.at[pl.ds(base, b_per_w)], idx_v.at[pl.ds(0, b_per_w)]
        )

        lanes = lax.iota(jnp.int32, _B)

        def body(b, carry):
            k0 = b * _RING
            v16 = idx_v[pl.ds(k0, _B)]
            copies = []
            for j in range(_RING):
                col0 = pl.multiple_of(
                    lax.shift_left(lax.shift_right_logical(v16[j], 7), 7),
                    128,
                )
                copies.append(
                    pltpu.async_copy(
                        tab_hbm.at[:, pl.ds(col0, 128)], ring_v.at[j], sem
                    )
                )
            for j in range(_RING):
                copies[j].wait()
                lsplat = plsc.load_gather(
                    idx_v, [jnp.full((_B,), k0 + j, jnp.int32)]
                )
                lsplat = lax.bitwise_and(lsplat, 127)
                for h in range(dim // _B):
                    vals = plsc.load_gather(
                        ring_v, [jnp.full((_B,), j, jnp.int32),
                                 lanes + h * _B, lsplat]
                    )
                    stage_v[k0 + j, pl.ds(h * _B, _B)] = vals
            return carry

        lax.fori_loop(0, b_per_w // _RING, body, 0)

        for j in range(dim):
            jsplat = jnp.full((_B,), j, jnp.int32)
            for c in range(b_per_w // _B):
                vals = plsc.load_gather(
                    stage_v, [lanes + c * _B, jsplat]
                )
                outt_v[j, pl.ds(c * _B, _B)] = vals
        pltpu.sync_copy(outt_v, out_hbm.at[:, pl.ds(base, b_per_w)])

    return lookup


@jax.jit
def kernel(skill_id, emb_weight):
    batch = skill_id.shape[0]
    n_rows, dim = emb_weight.shape
    out_t = _make_lookup(dim, batch)(
        skill_id.astype(jnp.int32), emb_weight.T
    )
    return out_t.T


# speedup vs baseline: 4.6449x; 1.5449x over previous
import functools

import jax
import jax.numpy as jnp
from jax import lax
from jax.experimental import pallas as pl
from jax.experimental.pallas import tpu as pltpu
from jax.experimental.pallas import tpu_sc as plsc

_INFO = plsc.get_sparse_core_info()
_NC = _INFO.num_cores
_NS = _INFO.num_subcores
_NW = _NC * _NS
_L = 16
_R = 8


def _make_lookup(dim, batch):
    assert batch % (_NW * 2 * _R) == 0
    b_per_w = batch // _NW
    n_pairs = b_per_w // (2 * _R)
    mesh = plsc.VectorSubcoreMesh(core_axis_name="c", subcore_axis_name="s")

    @functools.partial(
        pl.kernel,
        mesh=mesh,
        out_type=jax.ShapeDtypeStruct((dim, batch), jnp.float32),
        scratch_types=[
            pltpu.VMEM((b_per_w + _L,), jnp.int32),
            pltpu.VMEM((_R, dim, 128), jnp.float32),
            pltpu.VMEM((_R, dim, 128), jnp.float32),
            pltpu.VMEM((dim, b_per_w), jnp.float32),
            pltpu.SemaphoreType.DMA,
        ],
        compiler_params=pltpu.CompilerParams(needs_layout_passes=False),
    )
    def lookup(idx_hbm, tab_hbm, out_hbm, idx_v, ring_a, ring_b, outt_v,
               sem):
        wid = lax.axis_index("s") * _NC + lax.axis_index("c")
        base = wid * b_per_w
        pltpu.sync_copy(
            idx_hbm.at[pl.ds(base, b_per_w)], idx_v.at[pl.ds(0, b_per_w)]
        )

        lanes = lax.iota(jnp.int32, _L)

        def fire(k0, ring):
            v16 = idx_v[pl.ds(k0, _L)]
            for j in range(_R):
                col0 = pl.multiple_of(
                    lax.shift_left(
                        lax.shift_right_logical(v16[j], 7), 7
                    ),
                    128,
                )
                pltpu.async_copy(
                    tab_hbm.at[:, pl.ds(col0, 128)], ring.at[j], sem
                )

        def drain(ring):
            for j in range(_R):
                pltpu.make_async_copy(
                    tab_hbm.at[:, pl.ds(0, 128)], ring.at[j], sem
                ).wait()

        def extract(k0, ring):
            for j in range(_R):
                lsplat = plsc.load_gather(
                    idx_v, [jnp.full((_L,), k0 + j, jnp.int32)]
                )
                lsplat = lax.bitwise_and(lsplat, 127)
                ksplat = jnp.full((_L,), k0 + j, jnp.int32)
                jsplat = jnp.full((_L,), j, jnp.int32)
                for h in range(dim // _L):
                    vals = plsc.load_gather(
                        ring, [jsplat, lanes + h * _L, lsplat]
                    )
                    plsc.store_scatter(
                        outt_v, [lanes + h * _L, ksplat], vals
                    )

        def body(p, carry):
            k0 = p * 2 * _R
            fire(k0, ring_a)

            @pl.when(p > 0)
            def _prev():
                drain(ring_b)
                extract(k0 - _R, ring_b)

            fire(k0 + _R, ring_b)
            drain(ring_a)
            extract(k0, ring_a)
            return carry

        lax.fori_loop(0, n_pairs, body, 0)
        drain(ring_b)
        extract(b_per_w - _R, ring_b)

        pltpu.sync_copy(outt_v, out_hbm.at[:, pl.ds(base, b_per_w)])

    return lookup


@jax.jit
def kernel(skill_id, emb_weight):
    batch = skill_id.shape[0]
    n_rows, dim = emb_weight.shape
    out_t = _make_lookup(dim, batch)(
        skill_id.astype(jnp.int32), emb_weight.T
    )
    return out_t.T
